# Initial kernel scaffold; baseline (speedup 1.0000x reference)
import jax
import jax.numpy as jnp
from jax.experimental import pallas as pl


def kernel(pos_ids, emb, weight):
    raise NotImplementedError("write your pallas kernel here")


# speedup vs baseline: 1.1267x; 1.1267x over previous
import functools

import jax
import jax.numpy as jnp
from jax import lax
from jax.experimental import pallas as pl
from jax.experimental.pallas import tpu as pltpu
from jax.experimental.pallas import tpu_sc as plsc

NC = 2
NS = 16
NW = NC * NS

CHUNK = 16


def _scale_body(w_ref, emb_ref, out_ref):
    out_ref[...] = emb_ref[...] * w_ref[0]


@functools.partial(jax.jit, static_argnames=())
def _scale_table(weight, emb):
    v, d = emb.shape
    rows = 512
    return pl.pallas_call(
        _scale_body,
        grid=(v // rows,),
        in_specs=[
            pl.BlockSpec(memory_space=pltpu.SMEM),
            pl.BlockSpec((rows, d), lambda i: (i, 0)),
        ],
        out_specs=pl.BlockSpec((rows, d), lambda i: (i, 0)),
        out_shape=jax.ShapeDtypeStruct((v, d), jnp.float32),
    )(weight, emb)


def _make_gather(total, d):
    assert total % NW == 0
    b_per_w = total // NW
    assert b_per_w % CHUNK == 0
    nchunk = b_per_w // CHUNK
    mesh = plsc.VectorSubcoreMesh(core_axis_name="c", subcore_axis_name="s")

    @functools.partial(
        pl.kernel,
        mesh=mesh,
        out_type=jax.ShapeDtypeStruct((total, d), jnp.float32),
        scratch_types=[
            pltpu.VMEM((b_per_w,), jnp.int32),
            pltpu.VMEM((CHUNK, d), jnp.float32),
            pltpu.SemaphoreType.DMA,
        ],
    )
    def _k(table_hbm, idx_hbm, out_hbm, idx_v, buf, sem):
        wid = lax.axis_index("s") * NC + lax.axis_index("c")
        base = wid * b_per_w
        pltpu.sync_copy(idx_hbm.at[pl.ds(base, b_per_w)], idx_v)

        def body(c, carry):
            pltpu.async_copy(
                table_hbm.at[idx_v.at[pl.ds(c * CHUNK, CHUNK)]], buf, sem
            ).wait()
            pltpu.sync_copy(buf, out_hbm.at[pl.ds(base + c * CHUNK, CHUNK)])
            return carry

        lax.fori_loop(0, nchunk, body, 0)

    return _k


def kernel(pos_ids, emb, weight):
    v, d = emb.shape
    total = pos_ids.size
    scaled = _scale_table(weight.astype(jnp.float32), emb)
    idx = pos_ids.reshape(-1).astype(jnp.int32)
    out = _make_gather(total, d)(scaled, idx)
    return out.reshape(pos_ids.shape + (d,))


# speedup vs baseline: 1.3293x; 1.1798x over previous
import functools

import jax
import jax.numpy as jnp
from jax import lax
from jax.experimental import pallas as pl
from jax.experimental.pallas import tpu as pltpu
from jax.experimental.pallas import tpu_sc as plsc

NC = 2
NS = 16
NW = NC * NS

CHUNK = 16


def _scale_body(w_ref, emb_ref, out_ref):
    out_ref[...] = emb_ref[...] * w_ref[0]


@functools.partial(jax.jit, static_argnames=())
def _scale_table(weight, emb):
    v, d = emb.shape
    rows = 512
    return pl.pallas_call(
        _scale_body,
        grid=(v // rows,),
        in_specs=[
            pl.BlockSpec(memory_space=pltpu.SMEM),
            pl.BlockSpec((rows, d), lambda i: (i, 0)),
        ],
        out_specs=pl.BlockSpec((rows, d), lambda i: (i, 0)),
        out_shape=jax.ShapeDtypeStruct((v, d), jnp.float32),
    )(weight, emb)


def _make_gather(total, d):
    assert total % NW == 0
    b_per_w = total // NW
    assert b_per_w % CHUNK == 0
    nchunk = b_per_w // CHUNK
    assert nchunk % 2 == 0 and nchunk >= 4
    mesh = plsc.VectorSubcoreMesh(core_axis_name="c", subcore_axis_name="s")

    @functools.partial(
        pl.kernel,
        mesh=mesh,
        out_type=jax.ShapeDtypeStruct((total, d), jnp.float32),
        scratch_types=[
            pltpu.VMEM((b_per_w,), jnp.int32),
            pltpu.VMEM((CHUNK, d), jnp.float32),
            pltpu.VMEM((CHUNK, d), jnp.float32),
            pltpu.SemaphoreType.DMA,
            pltpu.SemaphoreType.DMA,
            pltpu.SemaphoreType.DMA,
            pltpu.SemaphoreType.DMA,
        ],
    )
    def _k(table_hbm, idx_hbm, out_hbm, idx_v, buf0, buf1, g0, g1, s0, s1):
        bufs = (buf0, buf1)
        gsems = (g0, g1)
        ssems = (s0, s1)
        wid = lax.axis_index("s") * NC + lax.axis_index("c")
        base = wid * b_per_w
        pltpu.sync_copy(idx_hbm.at[pl.ds(base, b_per_w)], idx_v)

        def gather(c, b):
            pltpu.async_copy(
                table_hbm.at[idx_v.at[pl.ds(c * CHUNK, CHUNK)]], bufs[b], gsems[b]
            )

        def wait_gather(c, b):
            pltpu.make_async_copy(
                table_hbm.at[idx_v.at[pl.ds(c * CHUNK, CHUNK)]], bufs[b], gsems[b]
            ).wait()

        def scatter(c, b):
            pltpu.async_copy(
                bufs[b], out_hbm.at[pl.ds(base + c * CHUNK, CHUNK)], ssems[b]
            )

        def wait_scatter(c, b):
            pltpu.make_async_copy(
                bufs[b], out_hbm.at[pl.ds(base + c * CHUNK, CHUNK)], ssems[b]
            ).wait()

        gather(0, 0)
        gather(1, 1)
        wait_gather(0, 0)
        scatter(0, 0)

        def body(i0, carry):
            for j in range(2):
                i = 1 + 2 * i0 + j
                b = (1 + j) % 2
                wait_scatter(i - 1, 1 - b)
                gather(i + 1, 1 - b)
                wait_gather(i, b)
                scatter(i, b)
            return carry

        lax.fori_loop(0, (nchunk - 2) // 2, body, 0)

        i = nchunk - 1
        wait_scatter(i - 1, 0)
        wait_gather(i, 1)
        scatter(i, 1)
        wait_scatter(i, 1)

    return _k


def kernel(pos_ids, emb, weight):
    v, d = emb.shape
    total = pos_ids.size
    scaled = _scale_table(weight.astype(jnp.float32), emb)
    idx = pos_ids.reshape(-1).astype(jnp.int32)
    out = _make_gather(total, d)(scaled, idx)
    return out.reshape(pos_ids.shape + (d,))


# speedup vs baseline: 1.5834x; 1.1912x over previous
import functools

import jax
import jax.numpy as jnp
from jax import lax
from jax.experimental import pallas as pl
from jax.experimental.pallas import tpu as pltpu
from jax.experimental.pallas import tpu_sc as plsc

NC = 2
NS = 16
NW = NC * NS

CHUNK = 16


def _make_fused_gather(total, d):
    assert total % NW == 0
    b_per_w = total // NW
    assert b_per_w % CHUNK == 0
    nchunk = b_per_w // CHUNK
    assert nchunk >= 5 and (nchunk - 4) % 3 == 0
    dv = d // 16
    assert dv & (dv - 1) == 0
    shift = dv.bit_length() - 1
    mesh = plsc.VectorSubcoreMesh(core_axis_name="c", subcore_axis_name="s")

    @functools.partial(
        pl.kernel,
        mesh=mesh,
        out_type=jax.ShapeDtypeStruct((total, d), jnp.float32),
        scratch_types=[
            pltpu.VMEM((b_per_w,), jnp.int32),
            pltpu.VMEM((16,), jnp.float32),
            pltpu.VMEM((CHUNK, d), jnp.float32),
            pltpu.VMEM((CHUNK, d), jnp.float32),
            pltpu.VMEM((CHUNK, d), jnp.float32),
            pltpu.SemaphoreType.DMA,
            pltpu.SemaphoreType.DMA,
            pltpu.SemaphoreType.DMA,
            pltpu.SemaphoreType.DMA,
            pltpu.SemaphoreType.DMA,
            pltpu.SemaphoreType.DMA,
        ],
    )
    def _k(table_hbm, idx_hbm, w_hbm, out_hbm,
           idx_v, w_v, buf0, buf1, buf2, g0, g1, g2, s0, s1, s2):
        bufs = (buf0, buf1, buf2)
        gsems = (g0, g1, g2)
        ssems = (s0, s1, s2)
        wid = lax.axis_index("s") * NC + lax.axis_index("c")
        base = wid * b_per_w
        pltpu.sync_copy(idx_hbm.at[pl.ds(base, b_per_w)], idx_v)
        pltpu.sync_copy(w_hbm, w_v)
        w = w_v[...]

        def gather(c, b):
            pltpu.async_copy(
                table_hbm.at[idx_v.at[pl.ds(c * CHUNK, CHUNK)]], bufs[b], gsems[b]
            )

        def wait_gather(c, b):
            pltpu.make_async_copy(
                table_hbm.at[idx_v.at[pl.ds(c * CHUNK, CHUNK)]], bufs[b], gsems[b]
            ).wait()

        def scatter(c, b):
            pltpu.async_copy(
                bufs[b], out_hbm.at[pl.ds(base + c * CHUNK, CHUNK)], ssems[b]
            )

        def wait_scatter(c, b):
            pltpu.make_async_copy(
                bufs[b], out_hbm.at[pl.ds(base + c * CHUNK, CHUNK)], ssems[b]
            ).wait()

        def scale(b):
            buf = bufs[b]

            @plsc.parallel_loop(0, CHUNK * dv, 1, unroll=8)
            def _(k):
                r = k >> shift
                col = (k & (dv - 1)) * 16
                buf[r, pl.ds(col, 16)] = buf[r, pl.ds(col, 16)] * w

        gather(0, 0)
        gather(1, 1)
        wait_gather(0, 0)
        scale(0)
        gather(2, 2)
        scatter(0, 0)
        wait_gather(1, 1)
        scale(1)

        def body(i0, carry):
            for j in range(3):
                i = 1 + 3 * i0 + j
                bm = (1 + j) % 3
                wait_scatter(i - 1, (bm + 2) % 3)
                gather(i + 2, (bm + 2) % 3)
                scatter(i, bm)
                wait_gather(i + 1, (bm + 1) % 3)
                scale((bm + 1) % 3)
            return carry

        lax.fori_loop(0, (nchunk - 4) // 3, body, 0)

        i = nchunk - 3
        wait_scatter(i - 1, 0)
        gather(i + 2, 0)
        scatter(i, 1)
        wait_gather(i + 1, 2)
        scale(2)
        i = nchunk - 2
        wait_scatter(i - 1, 1)
        scatter(i, 2)
        wait_gather(i + 1, 0)
        scale(0)
        i = nchunk - 1
        wait_scatter(i - 1, 2)
        scatter(i, 0)
        wait_scatter(i, 0)

    return _k


def kernel(pos_ids, emb, weight):
    v, d = emb.shape
    total = pos_ids.size
    idx = pos_ids.reshape(-1).astype(jnp.int32)
    w16 = jnp.broadcast_to(weight.astype(jnp.float32), (16,))
    out = _make_fused_gather(total, d)(emb, idx, w16)
    return out.reshape(pos_ids.shape + (d,))
